# baseline (speedup 1.0000x reference)
import jax
import jax.numpy as jnp
from jax import lax
from jax.experimental import pallas as pl
from jax.experimental.pallas import tpu as pltpu

OUT_H = 56
OUT_W = 56
S_DIM, N_DIM, C_DIM, HF, WF, M_DIM = 2, 8, 32, 128, 128, 64
B_BOX = 8


def _axis_weights(out_n, crop_len_f, crop_len_i, origin, size, transpose):
    if transpose:
        shape = (size, out_n)
        out_ax, pos_ax = 1, 0
    else:
        shape = (out_n, size)
        out_ax, pos_ax = 0, 1
    oi = lax.broadcasted_iota(jnp.int32, shape, out_ax).astype(jnp.float32)
    pos = lax.broadcasted_iota(jnp.int32, shape, pos_ax)
    scale = crop_len_f / out_n
    src = (oi + 0.5) * scale - 0.5
    src = jnp.maximum(src, 0.0)
    i0 = jnp.minimum(jnp.floor(src).astype(jnp.int32), crop_len_i - 1)
    i1 = jnp.minimum(i0 + 1, crop_len_i - 1)
    w = src - i0.astype(jnp.float32)
    return (jnp.where(pos == origin + i0, 1.0 - w, 0.0)
            + jnp.where(pos == origin + i1, w, 0.0))


def _roi_kernel(boxes_ref, fm_ref, out_ref, ft_ref, t1_ref, lhs2_ref):
    img = pl.program_id(0)
    jb = pl.program_id(1)

    @pl.when(jb == 0)
    def _():
        for c in range(C_DIM):
            ft_ref[:, c * WF:(c + 1) * WF] = fm_ref[0, c].astype(jnp.bfloat16)

    wy_list = []
    wxt_list = []
    for b in range(B_BOX):
        base = (img * M_DIM + jb * B_BOX + b) * 4
        x1 = boxes_ref[base + 0]
        y1 = boxes_ref[base + 1]
        x2 = boxes_ref[base + 2]
        y2 = boxes_ref[base + 3]
        ch_i = y2 - y1
        cw_i = x2 - x1
        wy_list.append(_axis_weights(OUT_H, ch_i.astype(jnp.float32), ch_i,
                                     y1, HF, transpose=False))
        wxt_list.append(_axis_weights(OUT_W, cw_i.astype(jnp.float32), cw_i,
                                      x1, WF, transpose=True))
    wy_all = jnp.concatenate(wy_list, axis=0).astype(jnp.bfloat16)

    t1_ref[...] = jnp.dot(wy_all, ft_ref[...],
                          preferred_element_type=jnp.float32
                          ).astype(jnp.bfloat16)

    for b in range(B_BOX):
        r0 = b * OUT_H
        for c in range(C_DIM):
            lhs2_ref[c * OUT_H:(c + 1) * OUT_H, :] = (
                t1_ref[r0:r0 + OUT_H, c * WF:(c + 1) * WF])
        out_ref[0, b] = jnp.dot(lhs2_ref[...],
                                wxt_list[b].astype(jnp.bfloat16),
                                preferred_element_type=jnp.float32
                                ).astype(jnp.bfloat16)


def kernel(feature_maps, boxes):
    S, N, C, Hf, Wf = feature_maps.shape
    M = boxes.shape[2]
    fm = feature_maps.reshape(S * N, C, Hf, Wf)
    boxes_flat = boxes.reshape(-1)

    grid = (S * N, M // B_BOX)
    out = pl.pallas_call(
        _roi_kernel,
        out_shape=jax.ShapeDtypeStruct((S, N * M, C * OUT_H, OUT_W),
                                       jnp.bfloat16),
        grid=grid,
        in_specs=[
            pl.BlockSpec(memory_space=pltpu.SMEM),
            pl.BlockSpec((1, C, Hf, Wf), lambda i, j: (i, 0, 0, 0)),
        ],
        out_specs=pl.BlockSpec(
            (1, B_BOX, C * OUT_H, OUT_W),
            lambda i, j: (i // N, (i % N) * (M // B_BOX) + j, 0, 0)),
        scratch_shapes=[
            pltpu.VMEM((Hf, C * Wf), jnp.bfloat16),
            pltpu.VMEM((B_BOX * OUT_H, C * Wf), jnp.bfloat16),
            pltpu.VMEM((C * OUT_H, Wf), jnp.bfloat16),
        ],
        compiler_params=pltpu.CompilerParams(
            dimension_semantics=("parallel", "arbitrary"),
            vmem_limit_bytes=50 * 1024 * 1024,
        ),
        name="roi_resize",
    )(boxes_flat, fm)
    return out.reshape(S, N * M, C, OUT_H, OUT_W).astype(jnp.float32)


# speedup vs baseline: 1.1845x; 1.1845x over previous
import jax
import jax.numpy as jnp
from jax import lax
from jax.experimental import pallas as pl
from jax.experimental.pallas import tpu as pltpu

OUT_H = 56
OUT_W = 56
S_DIM, N_DIM, C_DIM, HF, WF, M_DIM = 2, 8, 32, 128, 128, 64
B_BOX = 8


def _axis_weights(out_n, crop_len_f, crop_len_i, origin, size, transpose):
    if transpose:
        shape = (size, out_n)
        out_ax, pos_ax = 1, 0
    else:
        shape = (out_n, size)
        out_ax, pos_ax = 0, 1
    oi = lax.broadcasted_iota(jnp.int32, shape, out_ax).astype(jnp.float32)
    pos = lax.broadcasted_iota(jnp.int32, shape, pos_ax)
    scale = crop_len_f / out_n
    src = (oi + 0.5) * scale - 0.5
    src = jnp.maximum(src, 0.0)
    i0 = jnp.minimum(jnp.floor(src).astype(jnp.int32), crop_len_i - 1)
    i1 = jnp.minimum(i0 + 1, crop_len_i - 1)
    w = src - i0.astype(jnp.float32)
    return (jnp.where(pos == origin + i0, 1.0 - w, 0.0)
            + jnp.where(pos == origin + i1, w, 0.0))


def _roi_kernel(boxes_ref, fm_ref, out_ref, ft_ref, t1_ref, lhs2_ref):
    img = pl.program_id(0)
    jb = pl.program_id(1)

    @pl.when(jb == 0)
    def _():
        for c in range(C_DIM):
            ft_ref[:, c * WF:(c + 1) * WF] = fm_ref[0, c].astype(jnp.bfloat16)

    wy_list = []
    wxt_list = []
    for b in range(B_BOX):
        base = (img * M_DIM + jb * B_BOX + b) * 4
        x1 = boxes_ref[base + 0]
        y1 = boxes_ref[base + 1]
        x2 = boxes_ref[base + 2]
        y2 = boxes_ref[base + 3]
        ch_i = y2 - y1
        cw_i = x2 - x1
        wy_list.append(_axis_weights(OUT_H, ch_i.astype(jnp.float32), ch_i,
                                     y1, HF, transpose=False))
        wxt_list.append(_axis_weights(OUT_W, cw_i.astype(jnp.float32), cw_i,
                                      x1, WF, transpose=True))
    wy_all = jnp.concatenate(wy_list, axis=0).astype(jnp.bfloat16)

    t1_ref[...] = jnp.dot(wy_all, ft_ref[...],
                          preferred_element_type=jnp.float32
                          ).astype(jnp.bfloat16)

    for b in range(B_BOX):
        r0 = b * OUT_H
        for c in range(C_DIM):
            lhs2_ref[c * OUT_H:(c + 1) * OUT_H, :] = (
                t1_ref[r0:r0 + OUT_H, c * WF:(c + 1) * WF])
        out_ref[b] = jnp.dot(lhs2_ref[...],
                             wxt_list[b].astype(jnp.bfloat16),
                             preferred_element_type=jnp.float32)


def kernel(feature_maps, boxes):
    S, N, C, Hf, Wf = feature_maps.shape
    M = boxes.shape[2]

    call = pl.pallas_call(
        _roi_kernel,
        out_shape=jax.ShapeDtypeStruct((N * M, C * OUT_H, OUT_W),
                                       jnp.float32),
        grid=(N, M // B_BOX),
        in_specs=[
            pl.BlockSpec(memory_space=pltpu.SMEM),
            pl.BlockSpec((1, C, Hf, Wf), lambda i, j: (i, 0, 0, 0)),
        ],
        out_specs=pl.BlockSpec(
            (B_BOX, C * OUT_H, OUT_W),
            lambda i, j: (i * (M // B_BOX) + j, 0, 0)),
        scratch_shapes=[
            pltpu.VMEM((Hf, C * Wf), jnp.bfloat16),
            pltpu.VMEM((B_BOX * OUT_H, C * Wf), jnp.bfloat16),
            pltpu.VMEM((C * OUT_H, Wf), jnp.bfloat16),
        ],
        compiler_params=pltpu.CompilerParams(
            dimension_semantics=("parallel", "arbitrary"),
            vmem_limit_bytes=50 * 1024 * 1024,
        ),
        name="roi_resize",
    )
    halves = []
    for s in range(S):
        out_s = call(boxes[s].reshape(-1), feature_maps[s])
        halves.append(out_s.reshape(N * M, C, OUT_H, OUT_W))
    return jnp.stack(halves)


# speedup vs baseline: 1.6293x; 1.3756x over previous
import jax
import jax.numpy as jnp
from jax import lax
from jax.experimental import pallas as pl
from jax.experimental.pallas import tpu as pltpu

OUT_H = 56
OUT_W = 56
S_DIM, N_DIM, C_DIM, HF, WF, M_DIM = 2, 8, 32, 128, 128, 64
B_BOX = 16


def _axis_weights(out_n, crop_len_f, crop_len_i, origin, size, transpose):
    if transpose:
        shape = (size, out_n)
        out_ax, pos_ax = 1, 0
    else:
        shape = (out_n, size)
        out_ax, pos_ax = 0, 1
    oi = lax.broadcasted_iota(jnp.int32, shape, out_ax).astype(jnp.float32)
    pos = lax.broadcasted_iota(jnp.int32, shape, pos_ax)
    scale = crop_len_f / out_n
    src = (oi + 0.5) * scale - 0.5
    src = jnp.maximum(src, 0.0)
    i0 = jnp.minimum(jnp.floor(src).astype(jnp.int32), crop_len_i - 1)
    i1 = jnp.minimum(i0 + 1, crop_len_i - 1)
    w = src - i0.astype(jnp.float32)
    return (jnp.where(pos == origin + i0, 1.0 - w, 0.0)
            + jnp.where(pos == origin + i1, w, 0.0))


def _roi_kernel(boxes_ref, fm_ref, out_ref, ft_ref, t1_ref, lhs2_ref):
    img = pl.program_id(0)
    jb = pl.program_id(1)

    @pl.when(jb == 0)
    def _():
        for c in range(C_DIM):
            ft_ref[:, c * WF:(c + 1) * WF] = fm_ref[0, c].astype(jnp.bfloat16)

    wy_list = []
    wxt_list = []
    for b in range(B_BOX):
        base = (img * M_DIM + jb * B_BOX + b) * 4
        x1 = boxes_ref[base + 0]
        y1 = boxes_ref[base + 1]
        x2 = boxes_ref[base + 2]
        y2 = boxes_ref[base + 3]
        ch_i = y2 - y1
        cw_i = x2 - x1
        wy_list.append(_axis_weights(OUT_H, ch_i.astype(jnp.float32), ch_i,
                                     y1, HF, transpose=False))
        wxt_list.append(_axis_weights(OUT_W, cw_i.astype(jnp.float32), cw_i,
                                      x1, WF, transpose=True))
    wy_all = jnp.concatenate(wy_list, axis=0).astype(jnp.bfloat16)

    t1_ref[...] = jnp.dot(wy_all, ft_ref[...],
                          preferred_element_type=jnp.float32
                          ).astype(jnp.bfloat16)

    for b in range(B_BOX):
        r0 = b * OUT_H
        for c in range(C_DIM):
            lhs2_ref[c * OUT_H:(c + 1) * OUT_H, :] = (
                t1_ref[r0:r0 + OUT_H, c * WF:(c + 1) * WF])
        out_ref[0, b] = jnp.dot(lhs2_ref[...],
                                wxt_list[b].astype(jnp.bfloat16),
                                preferred_element_type=jnp.float32)


def kernel(feature_maps, boxes):
    S, N, C, Hf, Wf = feature_maps.shape
    M = boxes.shape[2]

    fm = feature_maps.reshape(S * N, C, Hf, Wf)
    boxes_flat = boxes.reshape(-1)

    grid = (S * N, M // B_BOX)
    out = pl.pallas_call(
        _roi_kernel,
        out_shape=jax.ShapeDtypeStruct((S, N * M, C * OUT_H, OUT_W),
                                       jnp.float32),
        grid=grid,
        in_specs=[
            pl.BlockSpec(memory_space=pltpu.SMEM),
            pl.BlockSpec((1, C, Hf, Wf), lambda i, j: (i, 0, 0, 0)),
        ],
        out_specs=pl.BlockSpec(
            (1, B_BOX, C * OUT_H, OUT_W),
            lambda i, j: (i // N, (i % N) * (M // B_BOX) + j, 0, 0)),
        scratch_shapes=[
            pltpu.VMEM((Hf, C * Wf), jnp.bfloat16),
            pltpu.VMEM((B_BOX * OUT_H, C * Wf), jnp.bfloat16),
            pltpu.VMEM((C * OUT_H, Wf), jnp.bfloat16),
        ],
        compiler_params=pltpu.CompilerParams(
            dimension_semantics=("parallel", "arbitrary"),
            vmem_limit_bytes=50 * 1024 * 1024,
        ),
        name="roi_resize",
    )(boxes_flat, fm)
    return out.reshape(S, N * M, C, OUT_H, OUT_W)
